# baseline (speedup 1.0000x reference)
import functools
import math

import jax
import jax.numpy as jnp
from jax import lax
from jax.experimental import pallas as pl
from jax.experimental.pallas import tpu as pltpu

_NUM_BINS = 512
_BIN_RES = 0.01
_T0 = 0.0
_HALF = _BIN_RES / 2
_NP = 4096
_CHUNK = 128
_NCHUNK = _NP // _CHUNK
_LOG2E = 1.4426950408889634
_SQ_HALF_PI = math.sqrt(0.5 / math.pi)


def _hist_kernel(scan_ref, fields_ref, out_ref, acc_ref, *, steps):
    j = pl.program_id(0)

    @pl.when(j == 0)
    def _():
        acc_ref[...] = jnp.zeros_like(acc_ref)

    r_bc = (lax.broadcasted_iota(jnp.int32, (_NUM_BINS, _CHUNK), 0) + 1
            ).astype(jnp.float32) * _HALF + (_T0 / 2)

    sx = scan_ref[0]
    sy = scan_ref[1]
    sz = scan_ref[2]

    acc = acc_ref[...]
    for c in range(_NCHUNK):
        f = fields_ref[:, c * _CHUNK:(c + 1) * _CHUNK]
        dx = f[0:1, :] - sx
        dy = f[1:2, :] - sy
        dz = f[2:3, :] - sz
        r0 = jnp.sqrt(dx * dx + dy * dy + dz * dz)
        colour = f[3:4, :]
        coefv = f[4:5, :]
        opac = f[5:6, :]
        scalev = f[6:7, :]
        sigma = jnp.maximum(jnp.exp(scalev), _HALF)
        isig = 1.0 / sigma
        coeff = 1.0 / (1.0 + jnp.exp(-coefv))
        amp = (opac * opac) * (colour * colour) * _HALF
        a_row = amp * coeff * _SQ_HALF_PI * isig
        b_row = amp * (1.0 - coeff) * (isig * isig)
        c2 = (-0.5 * _LOG2E) * (isig * isig)

        u = r_bc - r0
        q = u * u
        e = jnp.exp2(q * c2)
        w = a_row + b_row * u
        acc = acc + jnp.maximum(e * w, 0.0)
    acc_ref[...] = acc

    @pl.when(j == steps - 1)
    def _():
        r_col = (lax.broadcasted_iota(jnp.int32, (_NUM_BINS, 1), 0) + 1
                 ).astype(jnp.float32) * _HALF + (_T0 / 2)
        hist = jnp.sum(acc_ref[...], axis=1, keepdims=True)
        out_ref[:, :] = hist / (r_col * r_col)


def _run_shard(scan_point, fields):
    steps = fields.shape[1] // _NP
    out = pl.pallas_call(
        functools.partial(_hist_kernel, steps=steps),
        grid=(steps,),
        in_specs=[
            pl.BlockSpec(memory_space=pltpu.SMEM),
            pl.BlockSpec((8, _NP), lambda j: (0, j)),
        ],
        out_specs=pl.BlockSpec((_NUM_BINS, 1), lambda j: (0, 0)),
        out_shape=jax.ShapeDtypeStruct((_NUM_BINS, 1), jnp.float32),
        scratch_shapes=[pltpu.VMEM((_NUM_BINS, _CHUNK), jnp.float32)],
        compiler_params=pltpu.CompilerParams(
            dimension_semantics=("arbitrary",)),
    )(scan_point, fields)
    return out[:, 0]


def _fields_of(means, colours, coefficients, opacities, scales, view_id):
    n = means.shape[0]
    opac = jnp.take(opacities, view_id, axis=1)
    fields = jnp.stack([
        means[:, 0], means[:, 1], means[:, 2],
        colours[:, 0], coefficients[:, 0], opac, scales[:, 0],
    ], axis=0)
    npad = _NP * (-(-n // _NP))
    return jnp.pad(fields, ((0, 1), (0, npad - n)))


def kernel(means, scan_point, colours, coefficients, opacities, scales,
           view_id):
    devs = jax.devices()
    n = means.shape[0]
    if len(devs) >= 2 and n % 2 == 0:
        mesh = jax.sharding.Mesh(devs[:2], ("x",))
        P = jax.sharding.PartitionSpec

        def _shard_fn(means_l, scan_l, col_l, coef_l, opac_l, scale_l, vid):
            f = _fields_of(means_l, col_l, coef_l, opac_l, scale_l, vid)
            return jax.lax.psum(_run_shard(scan_l, f), "x")

        return jax.shard_map(
            _shard_fn, mesh=mesh,
            in_specs=(P("x"), P(), P("x"), P("x"), P("x"), P("x"), P()),
            out_specs=P(), check_vma=False,
        )(means, scan_point, colours, coefficients, opacities, scales,
          view_id)

    fields = _fields_of(means, colours, coefficients, opacities, scales,
                        view_id)
    return _run_shard(scan_point, fields)


# speedup vs baseline: 2.4358x; 2.4358x over previous
import functools
import math

import jax
import jax.numpy as jnp
from jax import lax
from jax.experimental import pallas as pl
from jax.experimental.pallas import tpu as pltpu

_NUM_BINS = 512
_BIN_RES = 0.01
_T0 = 0.0
_HALF = _BIN_RES / 2
_NP = 4096
_CHUNK = 128
_NCHUNK = _NP // _CHUNK
_LOG2E = 1.4426950408889634
_SQ_HALF_PI = math.sqrt(0.5 / math.pi)


def _hist_kernel(scan_ref, fields_ref, out_ref, acc_ref,
                 r0_ref, c2_ref, a_ref, b_ref, *, steps):
    j = pl.program_id(0)

    @pl.when(j == 0)
    def _():
        acc_ref[...] = jnp.zeros_like(acc_ref)

    r_bc = (lax.broadcasted_iota(jnp.int32, (_NUM_BINS, _CHUNK), 0) + 1
            ).astype(jnp.float32) * _HALF + (_T0 / 2)

    sx = scan_ref[0]
    sy = scan_ref[1]
    sz = scan_ref[2]

    dx = fields_ref[0:1, :] - sx
    dy = fields_ref[1:2, :] - sy
    dz = fields_ref[2:3, :] - sz
    r0 = jnp.sqrt(dx * dx + dy * dy + dz * dz)
    colour = fields_ref[3:4, :]
    coefv = fields_ref[4:5, :]
    opac = fields_ref[5:6, :]
    scalev = fields_ref[6:7, :]
    sigma = jnp.maximum(jnp.exp(scalev), _HALF)
    isig = 1.0 / sigma
    coeff = 1.0 / (1.0 + jnp.exp(-coefv))
    amp = (opac * opac) * (colour * colour) * _HALF
    r0_ref[...] = r0
    a_ref[...] = amp * coeff * _SQ_HALF_PI * isig
    b_ref[...] = amp * (1.0 - coeff) * (isig * isig)
    c2_ref[...] = (-0.5 * _LOG2E) * (isig * isig)

    acc = acc_ref[...]
    for c in range(_NCHUNK):
        sl = slice(c * _CHUNK, (c + 1) * _CHUNK)
        r0c = r0_ref[0:1, sl]
        c2c = c2_ref[0:1, sl]
        ac = a_ref[0:1, sl]
        bc = b_ref[0:1, sl]
        u = r_bc - r0c
        q = u * u
        e = jnp.exp2(q * c2c)
        w = ac + bc * u
        acc = acc + jnp.maximum(e * w, 0.0)
    acc_ref[...] = acc

    @pl.when(j == steps - 1)
    def _():
        r_col = (lax.broadcasted_iota(jnp.int32, (_NUM_BINS, 1), 0) + 1
                 ).astype(jnp.float32) * _HALF + (_T0 / 2)
        hist = jnp.sum(acc_ref[...], axis=1, keepdims=True)
        out_ref[:, :] = hist / (r_col * r_col)


def _run_shard(scan_point, fields):
    steps = fields.shape[1] // _NP
    out = pl.pallas_call(
        functools.partial(_hist_kernel, steps=steps),
        grid=(steps,),
        in_specs=[
            pl.BlockSpec(memory_space=pltpu.SMEM),
            pl.BlockSpec((8, _NP), lambda j: (0, j)),
        ],
        out_specs=pl.BlockSpec((_NUM_BINS, 1), lambda j: (0, 0)),
        out_shape=jax.ShapeDtypeStruct((_NUM_BINS, 1), jnp.float32),
        scratch_shapes=[pltpu.VMEM((_NUM_BINS, _CHUNK), jnp.float32),
                        pltpu.VMEM((1, _NP), jnp.float32),
                        pltpu.VMEM((1, _NP), jnp.float32),
                        pltpu.VMEM((1, _NP), jnp.float32),
                        pltpu.VMEM((1, _NP), jnp.float32)],
        compiler_params=pltpu.CompilerParams(
            dimension_semantics=("arbitrary",)),
    )(scan_point, fields)
    return out[:, 0]


def _fields_of(means, colours, coefficients, opacities, scales, view_id):
    n = means.shape[0]
    opac = jnp.take(opacities, view_id, axis=1)
    fields = jnp.stack([
        means[:, 0], means[:, 1], means[:, 2],
        colours[:, 0], coefficients[:, 0], opac, scales[:, 0],
    ], axis=0)
    npad = _NP * (-(-n // _NP))
    return jnp.pad(fields, ((0, 1), (0, npad - n)))


def kernel(means, scan_point, colours, coefficients, opacities, scales,
           view_id):
    fields = _fields_of(means, colours, coefficients, opacities, scales,
                        view_id)
    return _run_shard(scan_point, fields)
